# baseline (speedup 1.0000x reference)
import jax
import jax.numpy as jnp
from jax.experimental import pallas as pl


def _pe_kernel(x_ref, t_ref, o_ref):
    o_ref[...] = x_ref[...] + t_ref[...]


def kernel(x, table):
    batch, seq, dim = x.shape
    blk = 256
    return pl.pallas_call(
        _pe_kernel,
        grid=(batch // blk,),
        in_specs=[
            pl.BlockSpec((blk, seq, dim), lambda i: (i, 0, 0)),
            pl.BlockSpec((seq, dim), lambda i: (0, 0)),
        ],
        out_specs=pl.BlockSpec((blk, seq, dim), lambda i: (i, 0, 0)),
        out_shape=jax.ShapeDtypeStruct(x.shape, x.dtype),
    )(x, table)
